# Initial kernel scaffold; baseline (speedup 1.0000x reference)
import jax
import jax.numpy as jnp
from jax.experimental import pallas as pl


def kernel(x, edge_index, W_in, b_in, g_in, be_in, W1, b1, g1, be1, n0_g, n0_b, W2, b2, g2, be2, n1_g, n1_b, Wg1, bg1, Wg2, bg2, Wo, bo, go, beo):
    raise NotImplementedError("write your pallas kernel here")


# speedup vs baseline: 9.2734x; 9.2734x over previous
import functools

import jax
import jax.numpy as jnp
from jax import lax
from jax.experimental import pallas as pl
from jax.experimental.pallas import tpu as pltpu
from jax.experimental.pallas import tpu_sc as plsc

N = 10000
E = 320000
D = 128

NC = 2
NS = 16
NW = NC * NS
EPW = E // NW
CHUNK = 80
NCHUNK = EPW // CHUNK
ROWS_A = 624
ROWS_LAST = N - (NS - 1) * ROWS_A

BLK = 1000
GRID = N // BLK

_ZCOPIES = ROWS_A // CHUNK
_ZTAIL = ROWS_A - _ZCOPIES * CHUNK


def _zero_rows(buf, nrows, width):
    zeros16 = jnp.zeros((16,), jnp.float32)

    def _z(r, carry):
        for k in range(width // 16):
            buf[r, pl.ds(k * 16, 16)] = zeros16
        return carry
    lax.fori_loop(0, nrows, _z, 0)


def _zero_shared_slice(zsrc, shared, row0, s, sem):
    def _pairs():
        out = []
        for m in range(_ZCOPIES):
            out.append((zsrc, shared.at[pl.ds(row0 + m * CHUNK, CHUNK)]))
        out.append((zsrc.at[pl.ds(0, _ZTAIL)],
                    shared.at[pl.ds(row0 + _ZCOPIES * CHUNK, _ZTAIL)]))
        return out

    for a, b in _pairs():
        pltpu.async_copy(a, b, sem)

    tail = ROWS_LAST - ROWS_A

    @pl.when(s == NS - 1)
    def _():
        pltpu.async_copy(zsrc.at[pl.ds(0, tail)],
                         shared.at[pl.ds(row0 + ROWS_A, tail)], sem)

    for a, b in _pairs():
        pltpu.make_async_copy(a, b, sem).wait()

    @pl.when(s == NS - 1)
    def _():
        pltpu.make_async_copy(zsrc.at[pl.ds(0, tail)],
                              shared.at[pl.ds(row0 + ROWS_A, tail)],
                              sem).wait()


def _writeback_slice(shared, out, c, row0, s):
    for m in range(_ZCOPIES):
        r = row0 + m * CHUNK
        pltpu.sync_copy(shared.at[pl.ds(r, CHUNK)], out.at[c, pl.ds(r, CHUNK)])
    r = row0 + _ZCOPIES * CHUNK
    pltpu.sync_copy(shared.at[pl.ds(r, _ZTAIL)], out.at[c, pl.ds(r, _ZTAIL)])

    @pl.when(s == NS - 1)
    def _():
        tail = ROWS_LAST - ROWS_A
        pltpu.sync_copy(shared.at[pl.ds(row0 + ROWS_A, tail)],
                        out.at[c, pl.ds(row0 + ROWS_A, tail)])


def _sc_agg_body(h_hbm, src_hbm, dst_hbm, outp,
                 idx_src, idx_dst, rows, acc, gsem_a, gsem_b, ssem_a, ssem_b):
    gsems = (gsem_a, gsem_b)
    ssems = (ssem_a, ssem_b)

    c = lax.axis_index("c")
    s = lax.axis_index("s")
    wid = c * NS + s
    row0 = s * ROWS_A

    def _g(j, ring):
        return pltpu.async_copy(
            h_hbm.at[idx_src.at[pl.ds(j * CHUNK, CHUNK)]],
            rows.at[ring], gsems[ring])

    def _gwait(j, ring):
        pltpu.make_async_copy(
            h_hbm.at[idx_src.at[pl.ds(j * CHUNK, CHUNK)]],
            rows.at[ring], gsems[ring]).wait()

    def _s(j, ring):
        pltpu.async_copy(rows.at[ring], acc.at[idx_dst.at[j]], ssems[ring],
                         add=True)

    def _swait(j, ring):
        pltpu.make_async_copy(rows.at[ring], acc.at[idx_dst.at[j]],
                              ssems[ring]).wait()

    pltpu.sync_copy(src_hbm.at[wid], idx_src)
    pltpu.sync_copy(dst_hbm.at[wid], idx_dst)

    _g(0, 1)
    _zero_rows(rows.at[0], CHUNK, D)
    _zero_shared_slice(rows.at[0], acc, row0, s, gsems[0])
    plsc.subcore_barrier()
    _g(1, 0)

    def _pair(t, carry):
        j0 = 2 * t
        _gwait(j0, 1)
        _s(j0, 1)
        _gwait(j0 + 1, 0)
        _s(j0 + 1, 0)
        _swait(j0, 1)
        _g(j0 + 2, 1)
        _swait(j0 + 1, 0)
        _g(j0 + 3, 0)
        return carry
    lax.fori_loop(0, (NCHUNK - 3) // 2, _pair, 0)

    jl = NCHUNK - 3
    _gwait(jl, 1)
    _s(jl, 1)
    _gwait(jl + 1, 0)
    _s(jl + 1, 0)
    _swait(jl, 1)
    _g(jl + 2, 1)
    _swait(jl + 1, 0)
    _gwait(jl + 2, 1)
    _s(jl + 2, 1)
    _swait(jl + 2, 1)

    plsc.subcore_barrier()
    _writeback_slice(acc, outp, c, row0, s)


def _make_sc_agg():
    mesh = plsc.VectorSubcoreMesh(core_axis_name="c", subcore_axis_name="s")
    return pl.kernel(
        _sc_agg_body,
        [jax.ShapeDtypeStruct((NC, N, D), jnp.float32)],
        mesh=mesh,
        scratch_types=[
            pltpu.VMEM((EPW,), jnp.int32),
            pltpu.VMEM((NCHUNK, CHUNK), jnp.int32),
            pltpu.VMEM((2, CHUNK, D), jnp.float32),
            pltpu.VMEM_SHARED((N, D), jnp.float32),
            pltpu.SemaphoreType.DMA, pltpu.SemaphoreType.DMA,
            pltpu.SemaphoreType.DMA, pltpu.SemaphoreType.DMA,
        ],
    )


def _ln(y, g, b, eps=1e-5):
    m = jnp.mean(y, axis=-1, keepdims=True)
    v = jnp.mean((y - m) ** 2, axis=-1, keepdims=True)
    return (y - m) / jnp.sqrt(v + eps) * g + b


def _gelu(y):
    return 0.5 * y * (1.0 + lax.erf(y * (2.0 ** -0.5)))


def _dot(a, b):
    return jnp.dot(a, b, preferred_element_type=jnp.float32)


BIAS = 256.0


def _lastcol_mask():
    return lax.broadcasted_iota(jnp.int32, (BLK, D), 1) == (D - 1)


def _inproj_body(x_ref, w_ref, b_ref, g_ref, be_ref, o_ref):
    y = _dot(x_ref[...], w_ref[...]) + b_ref[...]
    h = _gelu(_ln(y, g_ref[...], be_ref[...]))
    o_ref[...] = h + jnp.where(_lastcol_mask(), BIAS, 0.0)


def _layer1_body(h_ref, p_ref, wt_ref, wb_ref,
                 b_ref, g_ref, be_ref, ng_ref, nb_ref, o_ref, cnt_ref):
    mask = _lastcol_mask()
    hb = h_ref[...]
    h = hb - jnp.where(mask, BIAS, 0.0)
    p = p_ref[...]
    P = p[0] + p[1]
    cntf = jnp.floor(P[:, D - 1] * (1.0 / BIAS) + 0.5)
    cnt = jnp.maximum(cntf, 1.0)
    cnt_ref[...] = cnt[:, None]
    agg = (P - jnp.where(mask, BIAS * cntf[:, None], 0.0)) / cnt[:, None]
    u = _dot(h, wt_ref[...]) + _dot(agg, wb_ref[...]) + b_ref[...]
    t = _ln(_gelu(u), g_ref[...], be_ref[...])
    t = _ln(t, ng_ref[...], nb_ref[...])
    o_ref[...] = _gelu(t) + h


def _layer2pool_body(h_ref, p_ref, c_ref, wt_ref, wb_ref,
                     b_ref, g_ref, be_ref, ng_ref, nb_ref,
                     wg1_ref, bg1_ref, wg2_ref, bg2_ref,
                     wo_ref, bo_ref, go_ref, beo_ref,
                     o_ref, emb_ref, gate_ref, h2_scr, lg_scr):
    ph = pl.program_id(0)
    i = pl.program_id(1)

    @pl.when(ph == 0)
    def _():
        cnt = c_ref[...][:, 0]
        p = p_ref[...]
        agg = (p[0] + p[1]) / cnt[:, None]
        h = h_ref[...]
        u = _dot(h, wt_ref[...]) + _dot(agg, wb_ref[...]) + b_ref[...]
        t = _ln(_gelu(u), g_ref[...], be_ref[...])
        h2 = _ln(t, ng_ref[...], nb_ref[...])
        o_ref[...] = h2
        h2_scr[pl.ds(i * BLK, BLK), :] = h2
        tg = jnp.tanh(_dot(h2, wg1_ref[...]) + bg1_ref[...])
        lg_scr[pl.ds(i * BLK, BLK), :] = _dot(tg, wg2_ref[...]) + bg2_ref[...]
        gate_ref[...] = jnp.zeros((BLK, 1), jnp.float32)

    @pl.when((ph == 1) & (i == 0))
    def _():
        lg = lg_scr[...]
        e = jnp.exp(lg - jnp.max(lg))
        gate = e / jnp.sum(e)
        lg_scr[...] = gate
        pooled = jnp.sum(h2_scr[...] * gate, axis=0, keepdims=True)
        y = _dot(pooled, wo_ref[...]) + bo_ref[...]
        emb_ref[...] = _gelu(_ln(y, go_ref[...], beo_ref[...]))

    @pl.when(ph == 1)
    def _():
        o_ref[...] = h2_scr[pl.ds(i * BLK, BLK), :]
        gate_ref[...] = lg_scr[pl.ds(i * BLK, BLK), :]


def _row_spec():
    return pl.BlockSpec((BLK, D), lambda i: (i, 0))


def _full2(shape):
    return pl.BlockSpec(shape, lambda i: tuple(0 for _ in shape))


_inproj = pl.pallas_call(
    _inproj_body,
    grid=(GRID,),
    in_specs=[_row_spec(), _full2((D, D)), _full2((D,)), _full2((D,)),
              _full2((D,))],
    out_specs=_row_spec(),
    out_shape=jax.ShapeDtypeStruct((N, D), jnp.float32),
)


_layer1 = pl.pallas_call(
    _layer1_body,
    grid=(GRID,),
    in_specs=[_row_spec(),
              pl.BlockSpec((NC, BLK, D), lambda i: (0, i, 0)),
              _full2((D, D)), _full2((D, D)), _full2((D,)), _full2((D,)),
              _full2((D,)), _full2((D,)), _full2((D,))],
    out_specs=[_row_spec(), pl.BlockSpec((BLK, 1), lambda i: (i, 0))],
    out_shape=[jax.ShapeDtypeStruct((N, D), jnp.float32),
               jax.ShapeDtypeStruct((N, 1), jnp.float32)],
)

def _fullp(shape):
    return pl.BlockSpec(shape, lambda p, i: tuple(0 for _ in shape))


_layer2pool = pl.pallas_call(
    _layer2pool_body,
    grid=(2, GRID),
    in_specs=[pl.BlockSpec((BLK, D), lambda p, i: ((1 - p) * i, 0)),
              pl.BlockSpec((NC, BLK, D), lambda p, i: (0, (1 - p) * i, 0)),
              pl.BlockSpec((BLK, 1), lambda p, i: ((1 - p) * i, 0)),
              _fullp((D, D)), _fullp((D, D)), _fullp((D,)), _fullp((D,)),
              _fullp((D,)), _fullp((D,)), _fullp((D,)),
              _fullp((D, D)), _fullp((D,)), _fullp((D, 1)), _fullp((1,)),
              _fullp((D, D)), _fullp((D,)), _fullp((D,)), _fullp((D,))],
    out_specs=[pl.BlockSpec((BLK, D), lambda p, i: (i, 0)),
               pl.BlockSpec((1, D), lambda p, i: (0, 0)),
               pl.BlockSpec((BLK, 1), lambda p, i: (i, 0))],
    out_shape=[jax.ShapeDtypeStruct((N, D), jnp.float32),
               jax.ShapeDtypeStruct((1, D), jnp.float32),
               jax.ShapeDtypeStruct((N, 1), jnp.float32)],
    scratch_shapes=[pltpu.VMEM((N, D), jnp.float32),
                    pltpu.VMEM((N, 1), jnp.float32)],
)

_sc_agg = _make_sc_agg()


def kernel(x, edge_index, W_in, b_in, g_in, be_in, W1, b1, g1, be1,
           n0_g, n0_b, W2, b2, g2, be2, n1_g, n1_b,
           Wg1, bg1, Wg2, bg2, Wo, bo, go, beo):
    src = edge_index[0].reshape(NW, EPW)
    dst = edge_index[1].reshape(NW, NCHUNK, CHUNK)

    h0b = _inproj(x, W_in, b_in, g_in, be_in)

    (p,) = _sc_agg(h0b, src, dst)
    h1, cnt = _layer1(h0b, p, W1[:D], W1[D:], b1, g1, be1, n0_g, n0_b)

    (q,) = _sc_agg(h1, src, dst)
    h2, emb, gate = _layer2pool(h1, q, cnt, W2[:D], W2[D:], b2, g2, be2,
                                n1_g, n1_b, Wg1, bg1, Wg2, bg2,
                                Wo, bo, go, beo)
    return (emb, h2, gate[:, 0])
